# baseline (speedup 1.0000x reference)
import functools

import jax
import jax.numpy as jnp
from jax import lax
from jax.experimental import pallas as pl
from jax.experimental.pallas import tpu as pltpu
from jax.experimental.pallas import tpu_sc as plsc

N = 50000
E = 800000
EMB = 64
HALF = 32
NCORES = 2
NTILES = 16
CHUNK = 128
NB = 5
NGRP = 2
STAGE = NB * NGRP
NSTG = 40
CPT = STAGE * NSTG
EP = NTILES * CPT * CHUNK
OWN = 3128
LAST = N - (NTILES - 1) * OWN
EXTRA = OWN - LAST
ZROWS = NB * CHUNK


def _swait(rows, acc, dst_st, sem, b):
    pltpu.make_async_copy(
        rows.at[pl.ds(b * CHUNK, CHUNK)], acc.at[dst_st.at[0, 0]], sem
    ).wait()


def _gwait(table, rows, src_st, sem, b):
    pltpu.make_async_copy(
        table.at[src_st.at[0, 0]], rows.at[pl.ds(b * CHUNK, CHUNK)], sem
    ).wait()


def _sc_body(table, src, dst, a, out, acc, src_st, dst_st, a_st, rows,
             g0, g1, g2, g3, g4, s0, s1, s2, s3, s4, t0):
    c = lax.axis_index("c")
    s = lax.axis_index("s")
    gsem = (g0, g1, g2, g3, g4)
    ssem = (s0, s1, s2, s3, s4)

    @pl.loop(0, ZROWS)
    def _zero(i):
        z = jnp.zeros((16,), jnp.float32)
        rows[i, pl.ds(0, 16)] = z
        rows[i, pl.ds(16, 16)] = z

    row0 = s * OWN
    nfull = LAST // ZROWS
    for kk in range(nfull):
        pltpu.sync_copy(rows, acc.at[pl.ds(row0 + kk * ZROWS, ZROWS)])
    rem = LAST - nfull * ZROWS
    pltpu.sync_copy(rows.at[pl.ds(0, rem)],
                    acc.at[pl.ds(row0 + nfull * ZROWS, rem)])

    @pl.when(s < NTILES - 1)
    def _zero_tail():
        pltpu.sync_copy(rows.at[pl.ds(0, EXTRA)],
                        acc.at[pl.ds(row0 + LAST, EXTRA)])

    plsc.subcore_barrier()

    tile_chunk0 = s * CPT
    coff = c * N

    def _gstart(sb, b, cidx):
        pltpu.async_copy(table.at[src_st.at[sb, cidx]],
                         rows.at[pl.ds(b * CHUNK, CHUNK)], gsem[b])

    def _sstart(sb, b, cidx):
        pltpu.async_copy(rows.at[pl.ds(b * CHUNK, CHUNK)],
                         acc.at[dst_st.at[sb, cidx]], ssem[b], add=True)

    def _scale(sb, b, cidx):
        base = b * CHUNK

        @pl.loop(0, CHUNK // 16, unroll=2)
        def _(jv):
            avec = a_st[sb, cidx, pl.ds(jv * 16, 16)]
            for lane in range(16):
                av = avec[lane]
                i = base + jv * 16 + lane
                rows[i, pl.ds(0, 16)] = rows[i, pl.ds(0, 16)] * av
                rows[i, pl.ds(16, 16)] = rows[i, pl.ds(16, 16)] * av

    def _stage_in(st):
        sb2 = st % 2
        base = tile_chunk0 + st * STAGE
        return (
            pltpu.async_copy(src.at[pl.ds(base, STAGE)], src_st.at[sb2], t0),
            pltpu.async_copy(dst.at[pl.ds(base, STAGE)], dst_st.at[sb2], t0),
            pltpu.async_copy(a.at[pl.ds(base, STAGE)], a_st.at[sb2], t0),
        )

    def _twait():
        pltpu.make_async_copy(src.at[pl.ds(0, STAGE)], src_st.at[0], t0).wait()

    def _shift(sb2):
        @pl.loop(0, STAGE)
        def _(k):
            for jj in range(8):
                v = src_st[sb2, k, pl.ds(jj * 16, 16)]
                src_st[sb2, k, pl.ds(jj * 16, 16)] = v + coff

    for d in _stage_in(0):
        d.wait()
    _stage_in(1)
    _shift(0)
    for b in range(NB - 1):
        _gstart(0, b, b)

    @pl.loop(0, NSTG)
    def _stage(st):
        sb = st % 2

        @pl.loop(0, NGRP)
        def _grp(grp):
            c0 = grp * NB
            for b in range(NB):
                bp = (b + NB - 1) % NB
                _gwait(table, rows, src_st, gsem[b], b)
                _scale(sb, b, c0 + b)
                _sstart(sb, b, c0 + b)
                if b == 0:
                    @pl.when(grp > 0)
                    def _():
                        _swait(rows, acc, dst_st, ssem[bp], bp)
                    _gstart(sb, bp, c0 + NB - 1)
                else:
                    @pl.when(grp < NGRP - 1)
                    def _():
                        _swait(rows, acc, dst_st, ssem[bp], bp)
                        _gstart(sb, bp, c0 + b + NB - 1)

        @pl.when(st < NSTG - 1)
        def _boundary():
            sb2 = (st + 1) % 2
            for _ in range(3):
                _twait()
            _shift(sb2)
            for b in range(NB - 1):
                _swait(rows, acc, dst_st, ssem[b], b)
                _gstart(sb2, b, b)

        @pl.when(st == NSTG - 1)
        def _last():
            for b in range(NB - 1):
                _swait(rows, acc, dst_st, ssem[b], b)

        _swait(rows, acc, dst_st, ssem[NB - 1], NB - 1)

        @pl.when(st < NSTG - 2)
        def _prefetch_stage():
            _stage_in(st + 2)

    plsc.subcore_barrier()
    pltpu.sync_copy(acc.at[pl.ds(row0, LAST)],
                    out.at[c].at[pl.ds(row0, LAST)])

    @pl.when(s < NTILES - 1)
    def _out_tail():
        pltpu.sync_copy(acc.at[pl.ds(row0 + LAST, EXTRA)],
                        out.at[c].at[pl.ds(row0 + LAST, EXTRA)])


@functools.cache
def _get_sc_pass():
    return pl.kernel(
        _sc_body,
        out_type=jax.ShapeDtypeStruct((NCORES, N, HALF), jnp.float32),
        mesh=plsc.VectorSubcoreMesh(core_axis_name="c", subcore_axis_name="s"),
        scratch_types=[
            pltpu.VMEM_SHARED((N, HALF), jnp.float32),
            pltpu.VMEM((2, STAGE, CHUNK), jnp.int32),
            pltpu.VMEM((2, STAGE, CHUNK), jnp.int32),
            pltpu.VMEM((2, STAGE, CHUNK), jnp.float32),
            pltpu.VMEM((ZROWS, HALF), jnp.float32),
        ] + [pltpu.SemaphoreType.DMA] * 11,
        compiler_params=pltpu.CompilerParams(use_tc_tiling_on_sc=False),
    )


def _sc_pass(table, src, dst, a):
    return _get_sc_pass()(table, src, dst, a)


Q = 500
GRID = (N * HALF) // 128 // Q


def _tc1_body(vf, cf, wv_lo, wv_hi, wc_lo, wc_hi, bv, bc, vt, ct):
    f = vf[0]
    vt[0, 0] = jnp.maximum(
        jnp.dot(f, wv_lo[...], preferred_element_type=jnp.float32) + bv[0, :], 0.0)
    vt[1, 0] = jnp.maximum(
        jnp.dot(f, wv_hi[...], preferred_element_type=jnp.float32) + bv[1, :], 0.0)
    g = cf[0]
    ct[0, 0] = jnp.maximum(
        jnp.dot(g, wc_lo[...], preferred_element_type=jnp.float32) + bc[0, :], 0.0)
    ct[1, 0] = jnp.maximum(
        jnp.dot(g, wc_hi[...], preferred_element_type=jnp.float32) + bc[1, :], 0.0)


def _update_half(e0, e1, a0, a1, ws, b):
    x = jnp.dot(e0, ws[0], preferred_element_type=jnp.float32)
    x += jnp.dot(e1, ws[1], preferred_element_type=jnp.float32)
    x += jnp.dot(a0, ws[2], preferred_element_type=jnp.float32)
    x += jnp.dot(a1, ws[3], preferred_element_type=jnp.float32)
    return jnp.maximum(x + b, 0.0)


def _tc2_body(ce, ag, wb, b, out):
    e0, e1, a0, a1 = ce[0, 0], ce[1, 0], ag[0, 0], ag[1, 0]
    out[0, 0] = _update_half(e0, e1, a0, a1, wb[0], b[0, :])
    out[1, 0] = _update_half(e0, e1, a0, a1, wb[1], b[1, :])


def _tc3_body(vt, ag, wb, b, wp, bp1, vlo, vhi, bp2, out):
    x_lo = _update_half(vt[0, 0], vt[1, 0], ag[0, 0], ag[1, 0], wb[0], b[0, :])
    x_hi = _update_half(vt[0, 0], vt[1, 0], ag[0, 0], ag[1, 0], wb[1], b[1, :])
    h_lo = jnp.maximum(
        jnp.dot(x_lo, wp[0], preferred_element_type=jnp.float32)
        + jnp.dot(x_hi, wp[1], preferred_element_type=jnp.float32) + bp1[0, :], 0.0)
    h_hi = jnp.maximum(
        jnp.dot(x_lo, wp[2], preferred_element_type=jnp.float32)
        + jnp.dot(x_hi, wp[3], preferred_element_type=jnp.float32) + bp1[1, :], 0.0)
    out[0] = (jnp.dot(h_lo, vlo[...], preferred_element_type=jnp.float32)
              + jnp.dot(h_hi, vhi[...], preferred_element_type=jnp.float32)
              + bp2[0, 0])


def _whole(shape):
    return pl.BlockSpec(shape, lambda i: (0,) * len(shape))


_HBLK = pl.BlockSpec((NCORES, 1, Q, 128), lambda i: (0, i, 0, 0))
_HSHAPE = jax.ShapeDtypeStruct((NCORES, GRID, Q, 128), jnp.float32)

_tc1 = pl.pallas_call(
    _tc1_body,
    grid=(GRID,),
    in_specs=[
        pl.BlockSpec((1, Q, 76), lambda i: (i, 0, 0)),
        pl.BlockSpec((1, Q, 20), lambda i: (i, 0, 0)),
        _whole((76, 128)),
        _whole((76, 128)),
        _whole((20, 128)),
        _whole((20, 128)),
        _whole((2, 128)),
        _whole((2, 128)),
    ],
    out_specs=[_HBLK, _HBLK],
    out_shape=[_HSHAPE, _HSHAPE],
)

_tc2 = pl.pallas_call(
    _tc2_body,
    grid=(GRID,),
    in_specs=[
        _HBLK,
        _HBLK,
        _whole((2, 4, 128, 128)),
        _whole((2, 128)),
    ],
    out_specs=_HBLK,
    out_shape=_HSHAPE,
)

_tc3 = pl.pallas_call(
    _tc3_body,
    grid=(GRID,),
    in_specs=[
        _HBLK,
        _HBLK,
        _whole((2, 4, 128, 128)),
        _whole((2, 128)),
        _whole((4, 128, 128)),
        _whole((2, 128)),
        _whole((128, 4)),
        _whole((128, 4)),
        _whole((1, 1)),
    ],
    out_specs=pl.BlockSpec((1, Q, 4), lambda i: (i, 0, 0)),
    out_shape=jax.ShapeDtypeStruct((GRID, Q, 4), jnp.float32),
)


def _bd4(m):
    return jnp.kron(jnp.eye(4, dtype=jnp.float32), m)


def _upd_weights(w_first, m1, m2):
    def half(h):
        sl = slice(h * HALF, (h + 1) * HALF)
        return jnp.stack([_bd4(w_first[:HALF, sl]), _bd4(w_first[HALF:, sl]),
                          _bd4(m1[:, sl]), _bd4(m2[:, sl])])
    return jnp.stack([half(0), half(1)])


def _tile4(b):
    return jnp.stack([jnp.tile(b[:HALF], 4), jnp.tile(b[HALF:], 4)])


def kernel(var_features, con_features, edge_index, edge_attr, W_var, b_var,
           W_con, b_con, W_edge, W_cu, b_cu, W_vu, b_vu, W_p1, b_p1, W_p2,
           b_p2):
    f32 = jnp.float32
    a = edge_attr.reshape(-1).astype(f32)
    con_idx = edge_index[0]
    var_idx = edge_index[1]

    pad = EP - E
    pad_idx = (jnp.arange(pad, dtype=jnp.int32) * 97) % N
    a_rs = jnp.concatenate([a, jnp.zeros((pad,), f32)]).reshape(-1, CHUNK)
    vp = jnp.concatenate([var_idx, pad_idx]).reshape(-1, CHUNK)
    cp = jnp.concatenate([con_idx, pad_idx]).reshape(-1, CHUNK)

    w = W_edge[:, 0]

    wv_t = W_var.T
    wc_t = W_con.T
    var_H, con_H = _tc1(
        var_features.reshape(GRID, Q, 76), con_features.reshape(GRID, Q, 20),
        _bd4(wv_t[:, :HALF]), _bd4(wv_t[:, HALF:]),
        _bd4(wc_t[:, :HALF]), _bd4(wc_t[:, HALF:]),
        _tile4(b_var), _tile4(b_con))

    agg1 = _sc_pass(var_H.reshape(NCORES * N, HALF), vp, cp, a_rs)

    m1c = w[:HALF, None] * W_cu[:, EMB:EMB + HALF].T
    m2c = w[HALF:, None] * W_cu[:, EMB + HALF:].T
    con_H2 = _tc2(con_H, agg1.reshape(_HSHAPE.shape),
                  _upd_weights(W_cu[:, :EMB].T, m1c, m2c), _tile4(b_cu))

    agg2 = _sc_pass(con_H2.reshape(NCORES * N, HALF), cp, vp, a_rs)

    m1v = w[:HALF, None] * W_vu[:, EMB:EMB + HALF].T
    m2v = w[HALF:, None] * W_vu[:, EMB + HALF:].T
    wp1_t = W_p1.T
    wp = jnp.stack([_bd4(wp1_t[:HALF, :HALF]), _bd4(wp1_t[HALF:, :HALF]),
                    _bd4(wp1_t[:HALF, HALF:]), _bd4(wp1_t[HALF:, HALF:])])
    wp2 = W_p2[0]
    vlo = jnp.kron(jnp.eye(4, dtype=f32), wp2[:HALF, None])
    vhi = jnp.kron(jnp.eye(4, dtype=f32), wp2[HALF:, None])
    scores = _tc3(var_H, agg2.reshape(_HSHAPE.shape),
                  _upd_weights(W_vu[:, :EMB].T, m1v, m2v), _tile4(b_vu),
                  wp, _tile4(b_p1), vlo, vhi, b_p2.reshape(1, 1))
    return scores.reshape(N)


# speedup vs baseline: 1.9099x; 1.9099x over previous
import functools

import jax
import jax.numpy as jnp
from jax import lax
from jax.experimental import pallas as pl
from jax.experimental.pallas import tpu as pltpu
from jax.experimental.pallas import tpu_sc as plsc

N = 50000
E = 800000
EMB = 64
HALF = 32
NCORES = 2
NTILES = 16
CHUNK = 128
NB = 5
NGRP = 2
STAGE = NB * NGRP
NSTG = 40
CPT = STAGE * NSTG
EP = NTILES * CPT * CHUNK
OWN = 3128
LAST = N - (NTILES - 1) * OWN
EXTRA = OWN - LAST
ZROWS = NB * CHUNK


def _swait(rows, acc, dst_st, sem, b):
    pltpu.make_async_copy(
        rows.at[pl.ds(b * CHUNK, CHUNK)], acc.at[dst_st.at[0, 0]], sem
    ).wait()


def _gwait(table, rows, src_st, sem, b):
    pltpu.make_async_copy(
        table.at[src_st.at[0, 0]], rows.at[pl.ds(b * CHUNK, CHUNK)], sem
    ).wait()


def _sc_body(table, src, dst, a, out, acc, src_st, dst_st, a_st, rows,
             g0, g1, g2, g3, g4, s0, s1, s2, s3, s4, t0):
    c = lax.axis_index("c")
    s = lax.axis_index("s")
    gsem = (g0, g1, g2, g3, g4)
    ssem = (s0, s1, s2, s3, s4)

    @pl.loop(0, ZROWS)
    def _zero(i):
        z = jnp.zeros((16,), jnp.float32)
        rows[i, pl.ds(0, 16)] = z
        rows[i, pl.ds(16, 16)] = z

    row0 = s * OWN
    nfull = LAST // ZROWS
    for kk in range(nfull):
        pltpu.sync_copy(rows, acc.at[pl.ds(row0 + kk * ZROWS, ZROWS)])
    rem = LAST - nfull * ZROWS
    pltpu.sync_copy(rows.at[pl.ds(0, rem)],
                    acc.at[pl.ds(row0 + nfull * ZROWS, rem)])

    @pl.when(s < NTILES - 1)
    def _zero_tail():
        pltpu.sync_copy(rows.at[pl.ds(0, EXTRA)],
                        acc.at[pl.ds(row0 + LAST, EXTRA)])

    plsc.subcore_barrier()

    tile_chunk0 = s * CPT
    coff = c * N

    def _gstart(sb, b, cidx):
        pltpu.async_copy(table.at[src_st.at[sb, cidx]],
                         rows.at[pl.ds(b * CHUNK, CHUNK)], gsem[b])

    def _sstart(sb, b, cidx):
        pltpu.async_copy(rows.at[pl.ds(b * CHUNK, CHUNK)],
                         acc.at[dst_st.at[sb, cidx]], ssem[b], add=True)

    def _scale(sb, b, cidx):
        base = b * CHUNK

        @pl.loop(0, CHUNK // 16)
        def _(jv):
            avec = a_st[sb, cidx, pl.ds(jv * 16, 16)]
            for lane in range(16):
                av = avec[lane]
                i = base + jv * 16 + lane
                rows[i, pl.ds(0, 16)] = rows[i, pl.ds(0, 16)] * av
                rows[i, pl.ds(16, 16)] = rows[i, pl.ds(16, 16)] * av

    def _stage_in(st):
        sb2 = st % 2
        base = tile_chunk0 + st * STAGE
        return (
            pltpu.async_copy(src.at[pl.ds(base, STAGE)], src_st.at[sb2], t0),
            pltpu.async_copy(dst.at[pl.ds(base, STAGE)], dst_st.at[sb2], t0),
            pltpu.async_copy(a.at[pl.ds(base, STAGE)], a_st.at[sb2], t0),
        )

    def _twait():
        pltpu.make_async_copy(src.at[pl.ds(0, STAGE)], src_st.at[0], t0).wait()

    def _shift(sb2):
        @pl.loop(0, STAGE)
        def _(k):
            for jj in range(8):
                v = src_st[sb2, k, pl.ds(jj * 16, 16)]
                src_st[sb2, k, pl.ds(jj * 16, 16)] = v + coff

    for d in _stage_in(0):
        d.wait()
    _stage_in(1)
    _shift(0)
    for b in range(NB - 1):
        _gstart(0, b, b)

    @pl.loop(0, NSTG)
    def _stage(st):
        sb = st % 2

        @pl.loop(0, NGRP)
        def _grp(grp):
            c0 = grp * NB
            for b in range(NB):
                bp = (b + NB - 1) % NB
                _gwait(table, rows, src_st, gsem[b], b)
                _scale(sb, b, c0 + b)
                _sstart(sb, b, c0 + b)
                if b == 0:
                    @pl.when(grp > 0)
                    def _():
                        _swait(rows, acc, dst_st, ssem[bp], bp)
                    _gstart(sb, bp, c0 + NB - 1)
                else:
                    @pl.when(grp < NGRP - 1)
                    def _():
                        _swait(rows, acc, dst_st, ssem[bp], bp)
                        _gstart(sb, bp, c0 + b + NB - 1)

        @pl.when(st < NSTG - 1)
        def _boundary():
            sb2 = (st + 1) % 2
            for _ in range(3):
                _twait()
            _shift(sb2)
            for b in range(NB - 1):
                _swait(rows, acc, dst_st, ssem[b], b)
                _gstart(sb2, b, b)

        @pl.when(st == NSTG - 1)
        def _last():
            for b in range(NB - 1):
                _swait(rows, acc, dst_st, ssem[b], b)

        _swait(rows, acc, dst_st, ssem[NB - 1], NB - 1)

        @pl.when(st < NSTG - 2)
        def _prefetch_stage():
            _stage_in(st + 2)

    plsc.subcore_barrier()
    pltpu.sync_copy(acc.at[pl.ds(row0, LAST)],
                    out.at[c].at[pl.ds(row0, LAST)])

    @pl.when(s < NTILES - 1)
    def _out_tail():
        pltpu.sync_copy(acc.at[pl.ds(row0 + LAST, EXTRA)],
                        out.at[c].at[pl.ds(row0 + LAST, EXTRA)])


@functools.cache
def _get_sc_pass():
    return pl.kernel(
        _sc_body,
        out_type=jax.ShapeDtypeStruct((NCORES, N, HALF), jnp.float32),
        mesh=plsc.VectorSubcoreMesh(core_axis_name="c", subcore_axis_name="s"),
        scratch_types=[
            pltpu.VMEM_SHARED((N, HALF), jnp.float32),
            pltpu.VMEM((2, STAGE, CHUNK), jnp.int32),
            pltpu.VMEM((2, STAGE, CHUNK), jnp.int32),
            pltpu.VMEM((2, STAGE, CHUNK), jnp.float32),
            pltpu.VMEM((ZROWS, HALF), jnp.float32),
        ] + [pltpu.SemaphoreType.DMA] * 11,
        compiler_params=pltpu.CompilerParams(use_tc_tiling_on_sc=False),
    )


def _sc_pass(table, src, dst, a):
    return _get_sc_pass()(table, src, dst, a)


Q = 500
GRID = (N * HALF) // 128 // Q


def _tc1_body(vf, cf, wv_lo, wv_hi, wc_lo, wc_hi, bv, bc, vt, ct):
    f = vf[0]
    vt[0, 0] = jnp.maximum(
        jnp.dot(f, wv_lo[...], preferred_element_type=jnp.float32) + bv[0, :], 0.0)
    vt[1, 0] = jnp.maximum(
        jnp.dot(f, wv_hi[...], preferred_element_type=jnp.float32) + bv[1, :], 0.0)
    g = cf[0]
    ct[0, 0] = jnp.maximum(
        jnp.dot(g, wc_lo[...], preferred_element_type=jnp.float32) + bc[0, :], 0.0)
    ct[1, 0] = jnp.maximum(
        jnp.dot(g, wc_hi[...], preferred_element_type=jnp.float32) + bc[1, :], 0.0)


def _update_half(e0, e1, a0, a1, ws, b):
    x = jnp.dot(e0, ws[0], preferred_element_type=jnp.float32)
    x += jnp.dot(e1, ws[1], preferred_element_type=jnp.float32)
    x += jnp.dot(a0, ws[2], preferred_element_type=jnp.float32)
    x += jnp.dot(a1, ws[3], preferred_element_type=jnp.float32)
    return jnp.maximum(x + b, 0.0)


def _tc2_body(ce, ag, wb, b, out):
    e0, e1, a0, a1 = ce[0, 0], ce[1, 0], ag[0, 0], ag[1, 0]
    out[0, 0] = _update_half(e0, e1, a0, a1, wb[0], b[0, :])
    out[1, 0] = _update_half(e0, e1, a0, a1, wb[1], b[1, :])


def _tc3_body(vt, ag, wb, b, wp, bp1, vlo, vhi, bp2, out):
    x_lo = _update_half(vt[0, 0], vt[1, 0], ag[0, 0], ag[1, 0], wb[0], b[0, :])
    x_hi = _update_half(vt[0, 0], vt[1, 0], ag[0, 0], ag[1, 0], wb[1], b[1, :])
    h_lo = jnp.maximum(
        jnp.dot(x_lo, wp[0], preferred_element_type=jnp.float32)
        + jnp.dot(x_hi, wp[1], preferred_element_type=jnp.float32) + bp1[0, :], 0.0)
    h_hi = jnp.maximum(
        jnp.dot(x_lo, wp[2], preferred_element_type=jnp.float32)
        + jnp.dot(x_hi, wp[3], preferred_element_type=jnp.float32) + bp1[1, :], 0.0)
    out[0] = (jnp.dot(h_lo, vlo[...], preferred_element_type=jnp.float32)
              + jnp.dot(h_hi, vhi[...], preferred_element_type=jnp.float32)
              + bp2[0, 0])


def _whole(shape):
    return pl.BlockSpec(shape, lambda i: (0,) * len(shape))


_HBLK = pl.BlockSpec((NCORES, 1, Q, 128), lambda i: (0, i, 0, 0))
_HSHAPE = jax.ShapeDtypeStruct((NCORES, GRID, Q, 128), jnp.float32)

_tc1 = pl.pallas_call(
    _tc1_body,
    grid=(GRID,),
    in_specs=[
        pl.BlockSpec((1, Q, 76), lambda i: (i, 0, 0)),
        pl.BlockSpec((1, Q, 20), lambda i: (i, 0, 0)),
        _whole((76, 128)),
        _whole((76, 128)),
        _whole((20, 128)),
        _whole((20, 128)),
        _whole((2, 128)),
        _whole((2, 128)),
    ],
    out_specs=[_HBLK, _HBLK],
    out_shape=[_HSHAPE, _HSHAPE],
)

_tc2 = pl.pallas_call(
    _tc2_body,
    grid=(GRID,),
    in_specs=[
        _HBLK,
        _HBLK,
        _whole((2, 4, 128, 128)),
        _whole((2, 128)),
    ],
    out_specs=_HBLK,
    out_shape=_HSHAPE,
)

_tc3 = pl.pallas_call(
    _tc3_body,
    grid=(GRID,),
    in_specs=[
        _HBLK,
        _HBLK,
        _whole((2, 4, 128, 128)),
        _whole((2, 128)),
        _whole((4, 128, 128)),
        _whole((2, 128)),
        _whole((128, 4)),
        _whole((128, 4)),
        _whole((1, 1)),
    ],
    out_specs=pl.BlockSpec((1, Q, 4), lambda i: (i, 0, 0)),
    out_shape=jax.ShapeDtypeStruct((GRID, Q, 4), jnp.float32),
)


def _bd4(m):
    return jnp.kron(jnp.eye(4, dtype=jnp.float32), m)


def _upd_weights(w_first, m1, m2):
    def half(h):
        sl = slice(h * HALF, (h + 1) * HALF)
        return jnp.stack([_bd4(w_first[:HALF, sl]), _bd4(w_first[HALF:, sl]),
                          _bd4(m1[:, sl]), _bd4(m2[:, sl])])
    return jnp.stack([half(0), half(1)])


def _tile4(b):
    return jnp.stack([jnp.tile(b[:HALF], 4), jnp.tile(b[HALF:], 4)])


def kernel(var_features, con_features, edge_index, edge_attr, W_var, b_var,
           W_con, b_con, W_edge, W_cu, b_cu, W_vu, b_vu, W_p1, b_p1, W_p2,
           b_p2):
    f32 = jnp.float32
    a = edge_attr.reshape(-1).astype(f32)
    con_idx = edge_index[0]
    var_idx = edge_index[1]

    pad = EP - E
    pad_idx = (jnp.arange(pad, dtype=jnp.int32) * 97) % N
    a_rs = jnp.concatenate([a, jnp.zeros((pad,), f32)]).reshape(-1, CHUNK)
    vp = jnp.concatenate([var_idx, pad_idx]).reshape(-1, CHUNK)
    cp = jnp.concatenate([con_idx, pad_idx]).reshape(-1, CHUNK)

    w = W_edge[:, 0]

    wv_t = W_var.T
    wc_t = W_con.T
    var_H, con_H = _tc1(
        var_features.reshape(GRID, Q, 76), con_features.reshape(GRID, Q, 20),
        _bd4(wv_t[:, :HALF]), _bd4(wv_t[:, HALF:]),
        _bd4(wc_t[:, :HALF]), _bd4(wc_t[:, HALF:]),
        _tile4(b_var), _tile4(b_con))

    agg1 = _sc_pass(var_H.reshape(NCORES * N, HALF), vp, cp, a_rs)

    m1c = w[:HALF, None] * W_cu[:, EMB:EMB + HALF].T
    m2c = w[HALF:, None] * W_cu[:, EMB + HALF:].T
    con_H2 = _tc2(con_H, agg1.reshape(_HSHAPE.shape),
                  _upd_weights(W_cu[:, :EMB].T, m1c, m2c), _tile4(b_cu))

    agg2 = _sc_pass(con_H2.reshape(NCORES * N, HALF), cp, vp, a_rs)

    m1v = w[:HALF, None] * W_vu[:, EMB:EMB + HALF].T
    m2v = w[HALF:, None] * W_vu[:, EMB + HALF:].T
    wp1_t = W_p1.T
    wp = jnp.stack([_bd4(wp1_t[:HALF, :HALF]), _bd4(wp1_t[HALF:, :HALF]),
                    _bd4(wp1_t[:HALF, HALF:]), _bd4(wp1_t[HALF:, HALF:])])
    wp2 = W_p2[0]
    vlo = jnp.kron(jnp.eye(4, dtype=f32), wp2[:HALF, None])
    vhi = jnp.kron(jnp.eye(4, dtype=f32), wp2[HALF:, None])
    scores = _tc3(var_H, agg2.reshape(_HSHAPE.shape),
                  _upd_weights(W_vu[:, :EMB].T, m1v, m2v), _tile4(b_vu),
                  wp, _tile4(b_p1), vlo, vhi, b_p2.reshape(1, 1))
    return scores.reshape(N)
